# baseline (speedup 1.0000x reference)
import jax
import jax.numpy as jnp
from jax import lax
from jax.experimental import pallas as pl
from jax.experimental.pallas import tpu as pltpu
from jax.experimental.pallas import tpu_sc as plsc

_VOCAB = 100000
_EMB = 64
_REGION = 5
_NW = 32
_CP = 98
_NVEC = _EMB // 16


def _sc_kernel(w_hbm, word_hbm, idx_hbm, out_hbm,
               idx_v, unit_v, word_v, out_v,
               isem0, isem1, gsem0, gsem1, ssem0, ssem1):
  B = idx_hbm.shape[0]
  rows_per_w = B // _NW
  wid = lax.axis_index("c") * 16 + lax.axis_index("s")
  b0 = wid * rows_per_w
  isems = (isem0, isem1)
  gsems = (gsem0, gsem1)
  ssems = (ssem0, ssem1)

  def fire_idx(b, h):
    pltpu.async_copy(idx_hbm.at[b, h], idx_v.at[h], isems[h])

  def wait_idx(h):
    pltpu.make_async_copy(idx_hbm.at[0, 0], idx_v.at[h], isems[h]).wait()

  def fire_gathers(h):
    for r in range(_REGION):
      pltpu.async_copy(w_hbm.at[idx_v.at[h, r]], unit_v.at[h, r], gsems[h])
    pltpu.async_copy(word_hbm.at[idx_v.at[h, _REGION]],
                     word_v.at[h], gsems[h])

  def wait_gathers(h):
    for r in range(_REGION):
      pltpu.make_async_copy(w_hbm.at[idx_v.at[h, r]],
                            unit_v.at[h, r], gsems[h]).wait()
    pltpu.make_async_copy(word_hbm.at[idx_v.at[h, _REGION]],
                          word_v.at[h], gsems[h]).wait()

  for h in range(2):
    fire_idx(b0, h)
  for h in range(2):
    wait_idx(h)
    fire_gathers(h)

  @pl.loop(0, rows_per_w, step=2)
  def _rows(jr):
    for rp in range(2):
      j = jr + rp
      b = b0 + j

      @pl.when(j >= 2)
      def _():
        pltpu.make_async_copy(out_v.at[rp], out_hbm.at[b], ssems[rp]).wait()

      for h in range(2):
        wait_gathers(h)

        @pl.when(j + 1 < rows_per_w)
        def _():
          fire_idx(b + 1, h)

        @pl.loop(0, _CP)
        def _pos(i):
          for k in range(_NVEC):
            sl = pl.ds(16 * k, 16)
            w = word_v[h, i, sl]
            acc = unit_v[h, 0, i, sl] * w
            for r in range(1, _REGION):
              acc = jnp.maximum(acc, unit_v[h, r, i, sl] * w)
            out_v[rp, h * _CP + i, sl] = acc

        @pl.when(j + 1 < rows_per_w)
        def _():
          wait_idx(h)
          fire_gathers(h)

      pltpu.async_copy(out_v.at[rp], out_hbm.at[b], ssems[rp])

  for rp in range(2):
    pltpu.make_async_copy(out_v.at[rp], out_hbm.at[b0], ssems[rp]).wait()


@jax.jit
def kernel(seq, W, word_table):
  B, L = seq.shape
  radius = _REGION // 2
  Lc = L - 2 * radius
  seq = seq.astype(jnp.int32)

  rows = [seq[:, r:r + Lc] + jnp.int32(r * _VOCAB) for r in range(_REGION)]
  rows.append(seq[:, radius:radius + Lc])
  idx = jnp.stack(rows, axis=1)
  idx = idx.reshape(B, _REGION + 1, 2, _CP).transpose(0, 2, 1, 3)

  mesh = plsc.VectorSubcoreMesh(core_axis_name="c", subcore_axis_name="s")
  run = pl.kernel(
      _sc_kernel,
      out_type=jax.ShapeDtypeStruct((B, Lc, _EMB), jnp.float32),
      mesh=mesh,
      compiler_params=pltpu.CompilerParams(use_tc_tiling_on_sc=False),
      scratch_types=[
          pltpu.VMEM((2, _REGION + 1, _CP), jnp.int32),
          pltpu.VMEM((2, _REGION, _CP, _EMB), jnp.float32),
          pltpu.VMEM((2, _CP, _EMB), jnp.float32),
          pltpu.VMEM((2, 2 * _CP, _EMB), jnp.float32),
          pltpu.SemaphoreType.DMA,
          pltpu.SemaphoreType.DMA,
          pltpu.SemaphoreType.DMA,
          pltpu.SemaphoreType.DMA,
          pltpu.SemaphoreType.DMA,
          pltpu.SemaphoreType.DMA,
      ],
  )
  return run(W, word_table, idx)
